# Initial kernel scaffold; baseline (speedup 1.0000x reference)
import jax
import jax.numpy as jnp
from jax.experimental import pallas as pl


def kernel(features, edge_index, noise, W1, b1, W2, b2, W3, b3):
    raise NotImplementedError("write your pallas kernel here")


# speedup vs baseline: 4.8924x; 4.8924x over previous
import functools

import jax
import jax.numpy as jnp
from jax import lax
from jax.experimental import pallas as pl
from jax.experimental.pallas import tpu as pltpu
from jax.experimental.pallas import tpu_sc as plsc

N = 10000
E = 160000
NP = 10240
EP = 163840
H1 = 32
H2 = 16

NC, NS = 2, 16
NW = NC * NS
ROWS = EP // 128
RPW = ROWS // NW
CH = 8
NCHUNK = RPW // CH
EPW = EP // NW
ZR = NP // NS

_SC_MESH = plsc.VectorSubcoreMesh(
    core_axis_name="c", subcore_axis_name="s", num_cores=NC, num_subcores=NS)
_SC_PARAMS = pltpu.CompilerParams(needs_layout_passes=False)
_SC_PARAMS_NOTILE = pltpu.CompilerParams(
    needs_layout_passes=False, use_tc_tiling_on_sc=False)


def _deg_body(src_hbm, dst_hbm, zero_hbm, out_hbm, dacc, sidx, didx):
    c = lax.axis_index("c")
    s = lax.axis_index("s")
    wid = s * NC + c
    base = wid * EPW
    pltpu.sync_copy(zero_hbm, dacc)
    pltpu.sync_copy(src_hbm.at[pl.ds(base, EPW)], sidx)
    pltpu.sync_copy(dst_hbm.at[pl.ds(base, EPW)], didx)
    ones_f = jnp.ones((16,), jnp.float32)
    off = jnp.full((16,), NP, jnp.int32)

    def step(i, carry):
        iv = sidx[pl.ds(i * 16, 16)]
        plsc.addupdate_scatter(dacc, [iv], ones_f)
        jv = didx[pl.ds(i * 16, 16)] + off
        plsc.addupdate_scatter(dacc, [jv], ones_f)
        return carry

    lax.fori_loop(0, EPW // 16, step, 0)
    pltpu.sync_copy(dacc, out_hbm.at[wid])


_deg_call = pl.kernel(
    _deg_body,
    out_type=jax.ShapeDtypeStruct((NW, 2 * NP), jnp.float32),
    mesh=_SC_MESH,
    compiler_params=_SC_PARAMS,
    scratch_types=[
        pltpu.VMEM((2 * NP,), jnp.float32),
        pltpu.VMEM((EPW,), jnp.int32),
        pltpu.VMEM((EPW,), jnp.int32),
    ],
)


def _agg_body(tbl_hbm, s2_hbm, d2_hbm, zrow_hbm, out_hbm,
              acc, sidx2, didx2, rows3, zbuf, gsem, ssem):
    c = lax.axis_index("c")
    s = lax.axis_index("s")
    wid = s * NC + c
    pltpu.sync_copy(zrow_hbm, zbuf)
    pltpu.sync_copy(zbuf, acc.at[pl.ds(s * ZR, ZR)])
    plsc.subcore_barrier()

    def chunk(i, carry):
        rb = wid * RPW + i * CH
        pltpu.sync_copy(s2_hbm.at[pl.ds(rb, CH)], sidx2)
        pltpu.sync_copy(d2_hbm.at[pl.ds(rb, CH)], didx2)
        gathers = [
            pltpu.async_copy(tbl_hbm.at[sidx2.at[j]], rows3.at[j], gsem)
            for j in range(CH)
        ]
        for g in gathers:
            g.wait()
        scatters = [
            pltpu.async_copy(rows3.at[j], acc.at[didx2.at[j]], ssem, add=True)
            for j in range(CH)
        ]
        for sc in scatters:
            sc.wait()
        return carry

    lax.fori_loop(0, NCHUNK, chunk, 0)
    plsc.subcore_barrier()
    pltpu.sync_copy(acc.at[pl.ds(s * ZR, ZR)], zbuf)
    pltpu.sync_copy(zbuf, out_hbm.at[c, pl.ds(s * ZR, ZR)])


_agg_call = pl.kernel(
    _agg_body,
    out_type=jax.ShapeDtypeStruct((NC, NP, H1), jnp.float32),
    mesh=_SC_MESH,
    compiler_params=_SC_PARAMS_NOTILE,
    scratch_types=[
        pltpu.VMEM_SHARED((NP, H1), jnp.float32),
        pltpu.VMEM((CH, 128), jnp.int32),
        pltpu.VMEM((CH, 128), jnp.int32),
        pltpu.VMEM((CH, 128, H1), jnp.float32),
        pltpu.VMEM((ZR, H1), jnp.float32),
        pltpu.SemaphoreType.DMA,
        pltpu.SemaphoreType.DMA,
    ],
)


_BR = 1280


def _tca_body(dp_ref, feat_ref, w1_ref, xwn_ref, norms_ref):
    deg = jnp.sum(dp_ref[...], axis=0)
    no = lax.rsqrt(jnp.maximum(deg[0], 1.0))
    ni = lax.rsqrt(jnp.maximum(deg[1], 1.0))
    xw = jnp.dot(feat_ref[...], w1_ref[...], preferred_element_type=jnp.float32)
    xwn_ref[...] = xw * no[:, None]
    norms_ref[...] = jnp.stack([no, ni])


def _tcb_body(ap_ref, norms_ref, b1_ref, hn_ref):
    a = ap_ref[0] + ap_ref[1]
    ni = norms_ref[1][:, None]
    no = norms_ref[0][:, None]
    hn_ref[...] = jnp.maximum(a * ni + b1_ref[...], 0.0) * no


def _tcc_body(ap_ref, norms_ref, noise_ref, w2_ref, b2_ref, w3_ref, b3_ref,
              z_ref):
    g = (ap_ref[0] + ap_ref[1]) * norms_ref[1][:, None]
    mean = jnp.dot(g, w2_ref[...], preferred_element_type=jnp.float32)
    log_std = jnp.dot(g, w3_ref[...], preferred_element_type=jnp.float32)
    z_ref[...] = (mean + b2_ref[...]
                  + noise_ref[...] * jnp.exp(log_std + b3_ref[...]))


_DR, _DC = 1000, 1280


def _tcd_body(z_ref, zt_ref, out_ref):
    x = jnp.dot(z_ref[...], zt_ref[...], preferred_element_type=jnp.float32)
    out_ref[...] = 0.5 * jnp.tanh(0.5 * x) + 0.5


def _tca(degp, featp, W1):
    grid = NP // _BR
    return pl.pallas_call(
        _tca_body,
        grid=(grid,),
        in_specs=[
            pl.BlockSpec((NW, 2, _BR), lambda i: (0, 0, i)),
            pl.BlockSpec((_BR, 128), lambda i: (i, 0)),
            pl.BlockSpec((128, H1), lambda i: (0, 0)),
        ],
        out_specs=[
            pl.BlockSpec((_BR, H1), lambda i: (i, 0)),
            pl.BlockSpec((2, _BR), lambda i: (0, i)),
        ],
        out_shape=[
            jax.ShapeDtypeStruct((NP, H1), jnp.float32),
            jax.ShapeDtypeStruct((2, NP), jnp.float32),
        ],
    )(degp, featp, W1)


def _tcb(aggp, norms, b1):
    grid = NP // _BR
    return pl.pallas_call(
        _tcb_body,
        grid=(grid,),
        in_specs=[
            pl.BlockSpec((NC, _BR, H1), lambda i: (0, i, 0)),
            pl.BlockSpec((2, _BR), lambda i: (0, i)),
            pl.BlockSpec((1, H1), lambda i: (0, 0)),
        ],
        out_specs=pl.BlockSpec((_BR, H1), lambda i: (i, 0)),
        out_shape=jax.ShapeDtypeStruct((NP, H1), jnp.float32),
    )(aggp, norms, b1)


def _tcc(aggp, norms, noisep, W2, b2, W3, b3):
    grid = NP // _BR
    return pl.pallas_call(
        _tcc_body,
        grid=(grid,),
        in_specs=[
            pl.BlockSpec((NC, _BR, H1), lambda i: (0, i, 0)),
            pl.BlockSpec((2, _BR), lambda i: (0, i)),
            pl.BlockSpec((_BR, H2), lambda i: (i, 0)),
            pl.BlockSpec((H1, H2), lambda i: (0, 0)),
            pl.BlockSpec((1, H2), lambda i: (0, 0)),
            pl.BlockSpec((H1, H2), lambda i: (0, 0)),
            pl.BlockSpec((1, H2), lambda i: (0, 0)),
        ],
        out_specs=pl.BlockSpec((_BR, H2), lambda i: (i, 0)),
        out_shape=jax.ShapeDtypeStruct((NP, H2), jnp.float32),
    )(aggp, norms, noisep, W2, b2, W3, b3)


def _tcd(zf, zt):
    return pl.pallas_call(
        _tcd_body,
        grid=(N // _DR, (N + _DC - 1) // _DC),
        in_specs=[
            pl.BlockSpec((_DR, H2), lambda i, j: (i, 0)),
            pl.BlockSpec((H2, _DC), lambda i, j: (0, j)),
        ],
        out_specs=pl.BlockSpec((_DR, _DC), lambda i, j: (i, j)),
        out_shape=jax.ShapeDtypeStruct((N, N), jnp.float32),
    )(zf, zt)


def kernel(features, edge_index, noise, W1, b1, W2, b2, W3, b3):
    src = edge_index[0]
    dst = edge_index[1]
    pad = EP - E
    srcp = jnp.concatenate([src, jnp.full((pad,), N, jnp.int32)])
    dstp = jnp.concatenate([dst, jnp.full((pad,), N, jnp.int32)])
    s2 = srcp.reshape(ROWS, 128)
    d2 = dstp.reshape(ROWS, 128)
    featp = jnp.pad(features, ((0, NP - N), (0, 0)))
    noisep = jnp.pad(noise, ((0, NP - N), (0, 0)))
    dzeros = jnp.zeros((2 * NP,), jnp.float32)
    zrows = jnp.zeros((ZR, H1), jnp.float32)

    degp = _deg_call(srcp, dstp, dzeros).reshape(NW, 2, NP)
    xwn, norms = _tca(degp, featp, W1)
    agg1 = _agg_call(xwn, s2, d2, zrows)
    hn = _tcb(agg1, norms, b1.reshape(1, H1))
    agg2 = _agg_call(hn, s2, d2, zrows)
    zf = _tcc(agg2, norms, noisep, W2, b2.reshape(1, H2),
              W3, b3.reshape(1, H2))
    zt = zf.T
    return _tcd(zf, zt)
